# baseline (speedup 1.0000x reference)
import functools
import jax
import jax.numpy as jnp
from jax import lax
from jax.experimental import pallas as pl
from jax.experimental.pallas import tpu as pltpu, tpu_sc as plsc

N = 10000
E = 320000
H = 128
G = 64
EPS = 1e-5

NP = 10240
NSUB = 16
NW = 2 * NSUB
CH = 96
EPT = 10176
EPAD = EPT * NW
NCH = EPT // CH
ROWS_PT = NP // NSUB


def _sc_mesh():
    return plsc.VectorSubcoreMesh(core_axis_name="c", subcore_axis_name="s",
                                  num_cores=2, num_subcores=NSUB)


def _sc_deg_body(dst_hbm, out_hbm, idx_v, deg_v):
    cid = lax.axis_index("c")
    sid = lax.axis_index("s")
    wid = cid * NSUB + sid

    def zb(i, c):
        deg_v[pl.ds(i * 16, 16)] = jnp.zeros((16,), jnp.float32)
        return c
    lax.fori_loop(0, NP // 16, zb, 0)

    pltpu.sync_copy(dst_hbm.at[pl.ds(wid * EPT, EPT)], idx_v)
    ones = jnp.ones((16,), jnp.float32)

    def eb(j, c):
        idx = idx_v[pl.ds(j * 16, 16)]
        plsc.addupdate_scatter(deg_v, [idx], ones)
        return c
    lax.fori_loop(0, EPT // 16, eb, 0)

    pltpu.sync_copy(deg_v, out_hbm.at[pl.ds(wid * NP, NP)])


@functools.cache
def _sc_deg_kernel():
    return pl.kernel(
        _sc_deg_body,
        out_type=jax.ShapeDtypeStruct((NW * NP,), jnp.float32),
        mesh=_sc_mesh(),
        scratch_types=[
            pltpu.VMEM((EPT,), jnp.int32),
            pltpu.VMEM((NP,), jnp.float32),
        ],
        compiler_params=pltpu.CompilerParams(needs_layout_passes=False),
    )


def _sc_deg(dstp):
    return _sc_deg_kernel()(dstp)


def _sc_scat_body(y_hbm, src_hbm, dst_hbm, out_hbm,
                  sidx_v, didx_v, sch0_v, dch0_v, sch1_v, dch1_v,
                  rows0_v, rows1_v, acc_sh, gsem0, gsem1, ssem0, ssem1):
    cid = lax.axis_index("c")
    sid = lax.axis_index("s")
    wid = cid * NSUB + sid
    base = wid * EPT

    def zb(i, c):
        r = i // (H // 16)
        k = i % (H // 16)
        rows0_v[r, pl.ds(k * 16, 16)] = jnp.zeros((16,), jnp.float32)
        return c
    lax.fori_loop(0, CH * (H // 16), zb, 0)

    def za(k, c):
        pltpu.sync_copy(rows0_v, acc_sh.at[pl.ds(sid * ROWS_PT + k * CH, CH)])
        return c
    lax.fori_loop(0, ROWS_PT // CH, za, 0)
    rem = ROWS_PT - (ROWS_PT // CH) * CH
    if rem:
        pltpu.sync_copy(
            rows0_v.at[pl.ds(0, rem)],
            acc_sh.at[pl.ds(sid * ROWS_PT + (ROWS_PT // CH) * CH, rem)])
    plsc.subcore_barrier()

    def cp(j, sch, dch):
        def cpk(k, c2):
            sch[pl.ds(k * 16, 16)] = sidx_v[pl.ds(j * CH + k * 16, 16)]
            dch[pl.ds(k * 16, 16)] = didx_v[pl.ds(j * CH + k * 16, 16)]
            return c2
        lax.fori_loop(0, CH // 16, cpk, 0)

    pltpu.sync_copy(src_hbm.at[pl.ds(base, EPT)], sidx_v)
    pltpu.sync_copy(dst_hbm.at[pl.ds(base, EPT)], didx_v)

    def wait_gather(sch, rows, sem):
        pltpu.make_async_copy(y_hbm.at[sch], rows, sem).wait()

    def wait_scatter(rows, dch, sem):
        pltpu.make_async_copy(rows, acc_sh.at[dch], sem).wait()

    cp(0, sch0_v, dch0_v)
    pltpu.async_copy(y_hbm.at[sch0_v], rows0_v, gsem0)

    def eb(t, c):
        j0 = 2 * t
        wait_gather(sch0_v, rows0_v, gsem0)
        pltpu.async_copy(rows0_v, acc_sh.at[dch0_v], ssem0, add=True)
        cp(j0 + 1, sch1_v, dch1_v)
        pltpu.async_copy(y_hbm.at[sch1_v], rows1_v, gsem1)
        wait_scatter(rows0_v, dch0_v, ssem0)

        @pl.when(t < NCH // 2 - 1)
        def _():
            cp(j0 + 2, sch0_v, dch0_v)
            pltpu.async_copy(y_hbm.at[sch0_v], rows0_v, gsem0)
        wait_gather(sch1_v, rows1_v, gsem1)
        pltpu.async_copy(rows1_v, acc_sh.at[dch1_v], ssem1, add=True)
        wait_scatter(rows1_v, dch1_v, ssem1)
        return c
    lax.fori_loop(0, NCH // 2, eb, 0)
    plsc.subcore_barrier()

    pltpu.sync_copy(acc_sh.at[pl.ds(sid * ROWS_PT, ROWS_PT)],
                    out_hbm.at[pl.ds(cid * NP + sid * ROWS_PT, ROWS_PT)])


@functools.cache
def _sc_scat_kernel():
    return pl.kernel(
        _sc_scat_body,
        out_type=jax.ShapeDtypeStruct((2 * NP, H), jnp.float32),
        mesh=_sc_mesh(),
        scratch_types=[
            pltpu.VMEM((EPT,), jnp.int32),
            pltpu.VMEM((EPT,), jnp.int32),
            pltpu.VMEM((CH,), jnp.int32),
            pltpu.VMEM((CH,), jnp.int32),
            pltpu.VMEM((CH,), jnp.int32),
            pltpu.VMEM((CH,), jnp.int32),
            pltpu.VMEM((CH, H), jnp.float32),
            pltpu.VMEM((CH, H), jnp.float32),
            pltpu.VMEM_SHARED((NP, H), jnp.float32),
            pltpu.SemaphoreType.DMA,
            pltpu.SemaphoreType.DMA,
            pltpu.SemaphoreType.DMA,
            pltpu.SemaphoreType.DMA,
        ],
        compiler_params=pltpu.CompilerParams(needs_layout_passes=False),
    )


def _sc_scat(y, srcp, dstp):
    return _sc_scat_kernel()(y, srcp, dstp)


def _lrelu(v):
    return jnp.where(v >= 0, v, 0.01 * v)


def _onehots(bcol, brow):
    iota_row = lax.broadcasted_iota(jnp.int32, (1, G), 1)
    iota_col = lax.broadcasted_iota(jnp.int32, (G, 1), 0)
    oh = (bcol == iota_row).astype(jnp.float32)
    oh_t = (brow == iota_col).astype(jnp.float32)
    return oh, oh_t


def _dot(a, b):
    return jnp.dot(a, b, preferred_element_type=jnp.float32)


def _dot_hi(a, b):
    return jnp.dot(a, b, preferred_element_type=jnp.float32,
                   precision=lax.Precision.HIGHEST)


BLK = 2000
BLKP = 2048
NBLK = 5

_NEG = -3.4e38


def _oh_of(bcol_blk):
    iota_row = lax.broadcasted_iota(jnp.int32, (1, G), 1)
    return (bcol_blk == iota_row).astype(jnp.float32)


def _dot_t(a, b):
    return lax.dot_general(a, b, (((0,), (0,)), ((), ())),
                           preferred_element_type=jnp.float32,
                           precision=lax.Precision.HIGHEST)


def _row_spec(w):
    return pl.BlockSpec((BLK, w), lambda i: (i, 0))


def _full_spec(hh, w):
    return pl.BlockSpec((hh, w), lambda i: (0, 0))


def _tc_preA_body(x_r, degp_r, w0_r, b0_r, h_r, dis_r):
    deg = jnp.sum(degp_r[...], axis=1, keepdims=True) + 1.0
    dis_r[...] = lax.rsqrt(deg)
    h_r[...] = _lrelu(_dot(x_r[...], w0_r[...]) + b0_r[...])


def _tc_preA(x, degp, w0, b0):
    return pl.pallas_call(
        _tc_preA_body,
        grid=(NBLK,),
        in_specs=[_row_spec(H),
                  pl.BlockSpec((BLKP, NW), lambda i: (i, 0)),
                  _full_spec(H, H), _full_spec(1, H)],
        out_specs=[_row_spec(H), pl.BlockSpec((BLKP, 1), lambda i: (i, 0))],
        out_shape=[jax.ShapeDtypeStruct((N, H), jnp.float32),
                   jax.ShapeDtypeStruct((NP, 1), jnp.float32)],
    )(x, degp, w0, b0)


def _tc_preB_body(bcol_r, h_r, gs_r, cnt_r):
    i = pl.program_id(0)
    oh = _oh_of(bcol_r[...])

    @pl.when(i == 0)
    def _():
        gs_r[...] = jnp.zeros((G, H), jnp.float32)
        cnt_r[...] = jnp.zeros((G, 1), jnp.float32)
    gs_r[...] += _dot_t(oh, h_r[...])
    cnt_r[...] += _dot_t(oh, jnp.ones((BLK, 1), jnp.float32))


def _tc_preB(bcol, h):
    return pl.pallas_call(
        _tc_preB_body,
        grid=(NBLK,),
        in_specs=[_row_spec(1), _row_spec(H)],
        out_specs=[_full_spec(G, H), _full_spec(G, 1)],
        out_shape=[jax.ShapeDtypeStruct((G, H), jnp.float32),
                   jax.ShapeDtypeStruct((G, 1), jnp.float32)],
    )(bcol, h)


def _tc_m3_body(h_r, bcol_r, dis_r, gap_r, gmp_r, cnt_r, pin_r, wc_r,
                y_r, pout_r):
    i = pl.program_id(0)
    cnt = cnt_r[...]
    gapv = gap_r[...] / jnp.maximum(cnt, 1.0)
    oh = _oh_of(bcol_r[...])
    glob = _dot_hi(oh, gapv)
    cat = jnp.concatenate([h_r[...], glob], axis=1)
    y_r[...] = dis_r[...] * _dot(cat, wc_r[...])

    @pl.when(i == 0)
    def _():
        gmpv = jnp.where(cnt > 0, gmp_r[...], 0.0)
        pout_r[...] = pin_r[...] + jnp.concatenate([gmpv, gapv], axis=1)


def _tc_m3(h, bcol, dis, gap, gmp, cnt, pooled, wc):
    return pl.pallas_call(
        _tc_m3_body,
        grid=(NBLK,),
        in_specs=[_row_spec(H), _row_spec(1), _row_spec(1),
                  _full_spec(G, H), _full_spec(G, H), _full_spec(G, 1),
                  _full_spec(G, 2 * H), _full_spec(2 * H, H)],
        out_specs=[_row_spec(H), _full_spec(G, 2 * H)],
        out_shape=[jax.ShapeDtypeStruct((NP, H), jnp.float32),
                   jax.ShapeDtypeStruct((G, 2 * H), jnp.float32)],
    )(h, bcol, dis, gap, gmp, cnt, pooled, wc)


def _tc_preC(h, bcol, dis, gs, cnt, wc):
    zero_gmp = jnp.zeros((G, H), jnp.float32)
    zero_pool = jnp.zeros((G, 2 * H), jnp.float32)
    y, _ = _tc_m3(h, bcol, dis, gs, zero_gmp, cnt, zero_pool, wc)
    return y


def _tc_m1_body(z0_r, z1_r, y_r, dis_r, bcol_r, bc_r, gcn_r, sg_r, s2_r):
    i = pl.program_id(0)
    gcn = dis_r[...] * (z0_r[...] + z1_r[...] + y_r[...]) + bc_r[...]
    gcn_r[...] = gcn
    oh = _oh_of(bcol_r[...])

    @pl.when(i == 0)
    def _():
        sg_r[...] = jnp.zeros((G, H), jnp.float32)
        s2_r[...] = jnp.zeros((G, 1), jnp.float32)
    sg_r[...] += _dot_t(oh, gcn)
    q = jnp.sum(gcn * gcn, axis=1, keepdims=True)
    s2_r[...] += _dot_t(oh, q)


def _tc_m1(z0, z1, y, dis, bcol, bc):
    return pl.pallas_call(
        _tc_m1_body,
        grid=(NBLK,),
        in_specs=[_row_spec(H), _row_spec(H), _row_spec(H), _row_spec(1),
                  _row_spec(1), _full_spec(1, H)],
        out_specs=[_row_spec(H), _full_spec(G, H), _full_spec(G, 1)],
        out_shape=[jax.ShapeDtypeStruct((N, H), jnp.float32),
                   jax.ShapeDtypeStruct((G, H), jnp.float32),
                   jax.ShapeDtypeStruct((G, 1), jnp.float32)],
    )(z0, z1, y, dis, bcol, bc)


def _tc_m2_body(gcn_r, bcol_r, sg_r, s2_r, cnt_r, g_r, bt_r,
                h_r, gap_r, gmp_r):
    i = pl.program_id(0)
    cnt = cnt_r[...]
    denom = jnp.maximum(cnt, 1.0) * H
    mean = jnp.sum(sg_r[...], axis=1, keepdims=True) / denom
    var = s2_r[...] / denom - mean * mean
    inv = lax.rsqrt(var + EPS)
    oh = _oh_of(bcol_r[...])
    mean_n = _dot_hi(oh, mean)
    inv_n = _dot_hi(oh, inv)
    xn = (gcn_r[...] - mean_n) * inv_n * g_r[...] + bt_r[...]
    h = _lrelu(xn)
    h_r[...] = h

    @pl.when(i == 0)
    def _():
        gap_r[...] = jnp.zeros((G, H), jnp.float32)
        gmp_r[...] = jnp.full((G, H), _NEG, jnp.float32)
    gap_r[...] += _dot_t(oh, h)

    bcol = bcol_r[...]
    iota_col = lax.broadcasted_iota(jnp.int32, (G, 1), 0)

    def body(g, acc):
        m = jnp.max(jnp.where(bcol == g, h, _NEG), axis=0, keepdims=True)
        return jnp.where(iota_col == g, jnp.maximum(m, acc), acc)
    gmp_r[...] = lax.fori_loop(0, G, body, gmp_r[...])


def _tc_m2(gcn, bcol, sg, s2, cnt, g, bt):
    return pl.pallas_call(
        _tc_m2_body,
        grid=(NBLK,),
        in_specs=[_row_spec(H),
                  _row_spec(1), _full_spec(G, H), _full_spec(G, 1),
                  _full_spec(G, 1), _full_spec(1, H), _full_spec(1, H)],
        out_specs=[_row_spec(H), _full_spec(G, H), _full_spec(G, H)],
        out_shape=[jax.ShapeDtypeStruct((N, H), jnp.float32),
                   jax.ShapeDtypeStruct((G, H), jnp.float32),
                   jax.ShapeDtypeStruct((G, H), jnp.float32)],
    )(gcn, bcol, sg, s2, cnt, g, bt)


def _tc_fin_body(gap_r, gmp_r, cnt_r, pin_r, w1_r, b1_r, w2_r, b2_r,
                 w3_r, b3_r, out_r):
    cnt = cnt_r[...]
    gapv = gap_r[...] / jnp.maximum(cnt, 1.0)
    gmpv = jnp.where(cnt > 0, gmp_r[...], 0.0)
    pooled = pin_r[...] + jnp.concatenate([gmpv, gapv], axis=1)
    o = _lrelu(_dot(pooled, w1_r[...]) + b1_r[...])
    o = _lrelu(_dot(o, w2_r[...]) + b2_r[...])
    out_r[...] = _dot(o, w3_r[...]) + b3_r[...]


def _tc_fin(gap, gmp, cnt, pooled, w1, b1, w2, b2, w3, b3):
    return pl.pallas_call(
        _tc_fin_body,
        out_shape=jax.ShapeDtypeStruct((G, 1), jnp.float32),
    )(gap, gmp, cnt, pooled, w1, b1, w2, b2, w3, b3)


@jax.jit
def kernel(x, edge_index, edge_attr, batch,
           W0, b0, Wc0, bc0, Wc1, bc1, Wc2, bc2,
           g0, bt0, g1, bt1, g2, bt2, W1, b1, W2, b2, W3, b3):
    del edge_attr
    src = edge_index[0]
    dst = edge_index[1]
    pad = jnp.full((EPAD - E,), N, jnp.int32)
    srcp = jnp.concatenate([src, pad])
    dstp = jnp.concatenate([dst, pad])

    bcol = batch.reshape(N, 1)

    degp = _sc_deg(dstp)
    degp_t = degp.reshape(NW, NP).T

    h, dis = _tc_preA(x, degp_t, W0, b0.reshape(1, H))
    gs, cnt = _tc_preB(bcol, h)
    y = _tc_preC(h, bcol, dis, gs, cnt, Wc0)

    pooled = jnp.zeros((G, 2 * H), jnp.float32)
    per_layer = [
        (bc0, g0, bt0, Wc1),
        (bc1, g1, bt1, Wc2),
    ]
    for bc, g, bt, wnext in per_layer:
        zf = _sc_scat(y, srcp, dstp).reshape(2, NP, H)
        gcn, sg, s2 = _tc_m1(zf[0], zf[1], y, dis, bcol, bc.reshape(1, H))
        h2, gap, gmp = _tc_m2(gcn, bcol, sg, s2, cnt,
                              g.reshape(1, H), bt.reshape(1, H))
        y, pooled = _tc_m3(h2, bcol, dis, gap, gmp, cnt, pooled, wnext)

    zf = _sc_scat(y, srcp, dstp).reshape(2, NP, H)
    gcn, sg, s2 = _tc_m1(zf[0], zf[1], y, dis, bcol, bc2.reshape(1, H))
    h2, gap, gmp = _tc_m2(gcn, bcol, sg, s2, cnt,
                          g2.reshape(1, H), bt2.reshape(1, H))
    out = _tc_fin(gap, gmp, cnt, pooled,
                  W1, b1.reshape(1, 4 * H), W2, b2.reshape(1, 4 * H),
                  W3, b3.reshape(1, 1))
    return out


# speedup vs baseline: 1.0306x; 1.0306x over previous
import functools
import jax
import jax.numpy as jnp
from jax import lax
from jax.experimental import pallas as pl
from jax.experimental.pallas import tpu as pltpu, tpu_sc as plsc

N = 10000
E = 320000
H = 128
G = 64
EPS = 1e-5

NP = 10240
NSUB = 16
NW = 2 * NSUB
CH = 128
EPT = 10112
EPAD = EPT * NW
NCH = EPT // CH
ROWS_PT = NP // NSUB


def _sc_mesh():
    return plsc.VectorSubcoreMesh(core_axis_name="c", subcore_axis_name="s",
                                  num_cores=2, num_subcores=NSUB)


def _sc_deg_body(dst_hbm, out_hbm, idx_v, deg_v):
    cid = lax.axis_index("c")
    sid = lax.axis_index("s")
    wid = cid * NSUB + sid

    def zb(i, c):
        deg_v[pl.ds(i * 16, 16)] = jnp.zeros((16,), jnp.float32)
        return c
    lax.fori_loop(0, NP // 16, zb, 0)

    pltpu.sync_copy(dst_hbm.at[pl.ds(wid * EPT, EPT)], idx_v)
    ones = jnp.ones((16,), jnp.float32)

    def eb(j, c):
        idx = idx_v[pl.ds(j * 16, 16)]
        plsc.addupdate_scatter(deg_v, [idx], ones)
        return c
    lax.fori_loop(0, EPT // 16, eb, 0)

    pltpu.sync_copy(deg_v, out_hbm.at[pl.ds(wid * NP, NP)])


@functools.cache
def _sc_deg_kernel():
    return pl.kernel(
        _sc_deg_body,
        out_type=jax.ShapeDtypeStruct((NW * NP,), jnp.float32),
        mesh=_sc_mesh(),
        scratch_types=[
            pltpu.VMEM((EPT,), jnp.int32),
            pltpu.VMEM((NP,), jnp.float32),
        ],
        compiler_params=pltpu.CompilerParams(needs_layout_passes=False),
    )


def _sc_deg(dstp):
    return _sc_deg_kernel()(dstp)


def _sc_scat_body(y_hbm, src_hbm, dst_hbm, out_hbm,
                  sidx_v, didx_v, sch0_v, dch0_v,
                  rows0_v, acc_sh, gsem0):
    cid = lax.axis_index("c")
    sid = lax.axis_index("s")
    wid = cid * NSUB + sid
    base = wid * EPT

    def zb(i, c):
        r = i // (H // 16)
        k = i % (H // 16)
        rows0_v[r, pl.ds(k * 16, 16)] = jnp.zeros((16,), jnp.float32)
        return c
    lax.fori_loop(0, CH * (H // 16), zb, 0)

    def za(k, c):
        pltpu.sync_copy(rows0_v, acc_sh.at[pl.ds(sid * ROWS_PT + k * CH, CH)])
        return c
    lax.fori_loop(0, ROWS_PT // CH, za, 0)
    plsc.subcore_barrier()

    def cp(j, sch, dch):
        def cpk(k, c2):
            sch[pl.ds(k * 16, 16)] = sidx_v[pl.ds(j * CH + k * 16, 16)]
            dch[pl.ds(k * 16, 16)] = didx_v[pl.ds(j * CH + k * 16, 16)]
            return c2
        lax.fori_loop(0, CH // 16, cpk, 0)

    pltpu.sync_copy(src_hbm.at[pl.ds(base, EPT)], sidx_v)
    pltpu.sync_copy(dst_hbm.at[pl.ds(base, EPT)], didx_v)

    def eb(j, c):
        cp(j, sch0_v, dch0_v)
        pltpu.async_copy(y_hbm.at[sch0_v], rows0_v, gsem0).wait()
        pltpu.sync_copy(rows0_v, acc_sh.at[dch0_v], add=True)
        return c
    lax.fori_loop(0, NCH, eb, 0)
    plsc.subcore_barrier()

    pltpu.sync_copy(acc_sh.at[pl.ds(sid * ROWS_PT, ROWS_PT)],
                    out_hbm.at[pl.ds(cid * NP + sid * ROWS_PT, ROWS_PT)])


@functools.cache
def _sc_scat_kernel():
    return pl.kernel(
        _sc_scat_body,
        out_type=jax.ShapeDtypeStruct((2 * NP, H), jnp.float32),
        mesh=_sc_mesh(),
        scratch_types=[
            pltpu.VMEM((EPT,), jnp.int32),
            pltpu.VMEM((EPT,), jnp.int32),
            pltpu.VMEM((CH,), jnp.int32),
            pltpu.VMEM((CH,), jnp.int32),
            pltpu.VMEM((CH, H), jnp.float32),
            pltpu.VMEM_SHARED((NP, H), jnp.float32),
            pltpu.SemaphoreType.DMA,
        ],
        compiler_params=pltpu.CompilerParams(needs_layout_passes=False),
    )


def _sc_scat(y, srcp, dstp):
    return _sc_scat_kernel()(y, srcp, dstp)


def _lrelu(v):
    return jnp.where(v >= 0, v, 0.01 * v)


def _onehots(bcol, brow):
    iota_row = lax.broadcasted_iota(jnp.int32, (1, G), 1)
    iota_col = lax.broadcasted_iota(jnp.int32, (G, 1), 0)
    oh = (bcol == iota_row).astype(jnp.float32)
    oh_t = (brow == iota_col).astype(jnp.float32)
    return oh, oh_t


def _dot(a, b):
    return jnp.dot(a, b, preferred_element_type=jnp.float32)


def _dot_hi(a, b):
    return jnp.dot(a, b, preferred_element_type=jnp.float32,
                   precision=lax.Precision.HIGHEST)


BLK = 2000
BLKP = 2048
NBLK = 5

_NEG = -3.4e38


def _oh_of(bcol_blk):
    iota_row = lax.broadcasted_iota(jnp.int32, (1, G), 1)
    return (bcol_blk == iota_row).astype(jnp.float32)


def _dot_t(a, b):
    return lax.dot_general(a, b, (((0,), (0,)), ((), ())),
                           preferred_element_type=jnp.float32,
                           precision=lax.Precision.HIGHEST)


def _row_spec(w):
    return pl.BlockSpec((BLK, w), lambda i: (i, 0))


def _full_spec(hh, w):
    return pl.BlockSpec((hh, w), lambda i: (0, 0))


def _tc_preA_body(x_r, degp_r, w0_r, b0_r, h_r, dis_r):
    deg = jnp.sum(degp_r[...], axis=1, keepdims=True) + 1.0
    dis_r[...] = lax.rsqrt(deg)
    h_r[...] = _lrelu(_dot(x_r[...], w0_r[...]) + b0_r[...])


def _tc_preA(x, degp, w0, b0):
    return pl.pallas_call(
        _tc_preA_body,
        grid=(NBLK,),
        in_specs=[_row_spec(H),
                  pl.BlockSpec((BLKP, NW), lambda i: (i, 0)),
                  _full_spec(H, H), _full_spec(1, H)],
        out_specs=[_row_spec(H), pl.BlockSpec((BLKP, 1), lambda i: (i, 0))],
        out_shape=[jax.ShapeDtypeStruct((N, H), jnp.float32),
                   jax.ShapeDtypeStruct((NP, 1), jnp.float32)],
    )(x, degp, w0, b0)


def _tc_preB_body(bcol_r, h_r, gs_r, cnt_r):
    i = pl.program_id(0)
    oh = _oh_of(bcol_r[...])

    @pl.when(i == 0)
    def _():
        gs_r[...] = jnp.zeros((G, H), jnp.float32)
        cnt_r[...] = jnp.zeros((G, 1), jnp.float32)
    gs_r[...] += _dot_t(oh, h_r[...])
    cnt_r[...] += _dot_t(oh, jnp.ones((BLK, 1), jnp.float32))


def _tc_preB(bcol, h):
    return pl.pallas_call(
        _tc_preB_body,
        grid=(NBLK,),
        in_specs=[_row_spec(1), _row_spec(H)],
        out_specs=[_full_spec(G, H), _full_spec(G, 1)],
        out_shape=[jax.ShapeDtypeStruct((G, H), jnp.float32),
                   jax.ShapeDtypeStruct((G, 1), jnp.float32)],
    )(bcol, h)


def _tc_m3_body(h_r, bcol_r, dis_r, gap_r, gmp_r, cnt_r, pin_r, wc_r,
                y_r, pout_r):
    i = pl.program_id(0)
    cnt = cnt_r[...]
    gapv = gap_r[...] / jnp.maximum(cnt, 1.0)
    oh = _oh_of(bcol_r[...])
    glob = _dot_hi(oh, gapv)
    cat = jnp.concatenate([h_r[...], glob], axis=1)
    y_r[...] = dis_r[...] * _dot(cat, wc_r[...])

    @pl.when(i == 0)
    def _():
        gmpv = jnp.where(cnt > 0, gmp_r[...], 0.0)
        pout_r[...] = pin_r[...] + jnp.concatenate([gmpv, gapv], axis=1)


def _tc_m3(h, bcol, dis, gap, gmp, cnt, pooled, wc):
    return pl.pallas_call(
        _tc_m3_body,
        grid=(NBLK,),
        in_specs=[_row_spec(H), _row_spec(1), _row_spec(1),
                  _full_spec(G, H), _full_spec(G, H), _full_spec(G, 1),
                  _full_spec(G, 2 * H), _full_spec(2 * H, H)],
        out_specs=[_row_spec(H), _full_spec(G, 2 * H)],
        out_shape=[jax.ShapeDtypeStruct((NP, H), jnp.float32),
                   jax.ShapeDtypeStruct((G, 2 * H), jnp.float32)],
    )(h, bcol, dis, gap, gmp, cnt, pooled, wc)


def _tc_preC(h, bcol, dis, gs, cnt, wc):
    zero_gmp = jnp.zeros((G, H), jnp.float32)
    zero_pool = jnp.zeros((G, 2 * H), jnp.float32)
    y, _ = _tc_m3(h, bcol, dis, gs, zero_gmp, cnt, zero_pool, wc)
    return y


def _tc_m1_body(z0_r, z1_r, y_r, dis_r, bcol_r, bc_r, gcn_r, sg_r, s2_r):
    i = pl.program_id(0)
    gcn = dis_r[...] * (z0_r[...] + z1_r[...] + y_r[...]) + bc_r[...]
    gcn_r[...] = gcn
    oh = _oh_of(bcol_r[...])

    @pl.when(i == 0)
    def _():
        sg_r[...] = jnp.zeros((G, H), jnp.float32)
        s2_r[...] = jnp.zeros((G, 1), jnp.float32)
    sg_r[...] += _dot_t(oh, gcn)
    q = jnp.sum(gcn * gcn, axis=1, keepdims=True)
    s2_r[...] += _dot_t(oh, q)


def _tc_m1(z0, z1, y, dis, bcol, bc):
    return pl.pallas_call(
        _tc_m1_body,
        grid=(NBLK,),
        in_specs=[_row_spec(H), _row_spec(H), _row_spec(H), _row_spec(1),
                  _row_spec(1), _full_spec(1, H)],
        out_specs=[_row_spec(H), _full_spec(G, H), _full_spec(G, 1)],
        out_shape=[jax.ShapeDtypeStruct((N, H), jnp.float32),
                   jax.ShapeDtypeStruct((G, H), jnp.float32),
                   jax.ShapeDtypeStruct((G, 1), jnp.float32)],
    )(z0, z1, y, dis, bcol, bc)


def _tc_m2_body(gcn_r, bcol_r, sg_r, s2_r, cnt_r, g_r, bt_r,
                h_r, gap_r, gmp_r):
    i = pl.program_id(0)
    cnt = cnt_r[...]
    denom = jnp.maximum(cnt, 1.0) * H
    mean = jnp.sum(sg_r[...], axis=1, keepdims=True) / denom
    var = s2_r[...] / denom - mean * mean
    inv = lax.rsqrt(var + EPS)
    oh = _oh_of(bcol_r[...])
    mean_n = _dot_hi(oh, mean)
    inv_n = _dot_hi(oh, inv)
    xn = (gcn_r[...] - mean_n) * inv_n * g_r[...] + bt_r[...]
    h = _lrelu(xn)
    h_r[...] = h

    @pl.when(i == 0)
    def _():
        gap_r[...] = jnp.zeros((G, H), jnp.float32)
        gmp_r[...] = jnp.full((G, H), _NEG, jnp.float32)
    gap_r[...] += _dot_t(oh, h)

    bcol = bcol_r[...]
    iota_col = lax.broadcasted_iota(jnp.int32, (G, 1), 0)

    def body(g, acc):
        m = jnp.max(jnp.where(bcol == g, h, _NEG), axis=0, keepdims=True)
        return jnp.where(iota_col == g, jnp.maximum(m, acc), acc)
    gmp_r[...] = lax.fori_loop(0, G, body, gmp_r[...])


def _tc_m2(gcn, bcol, sg, s2, cnt, g, bt):
    return pl.pallas_call(
        _tc_m2_body,
        grid=(NBLK,),
        in_specs=[_row_spec(H),
                  _row_spec(1), _full_spec(G, H), _full_spec(G, 1),
                  _full_spec(G, 1), _full_spec(1, H), _full_spec(1, H)],
        out_specs=[_row_spec(H), _full_spec(G, H), _full_spec(G, H)],
        out_shape=[jax.ShapeDtypeStruct((N, H), jnp.float32),
                   jax.ShapeDtypeStruct((G, H), jnp.float32),
                   jax.ShapeDtypeStruct((G, H), jnp.float32)],
    )(gcn, bcol, sg, s2, cnt, g, bt)


def _tc_fin_body(gap_r, gmp_r, cnt_r, pin_r, w1_r, b1_r, w2_r, b2_r,
                 w3_r, b3_r, out_r):
    cnt = cnt_r[...]
    gapv = gap_r[...] / jnp.maximum(cnt, 1.0)
    gmpv = jnp.where(cnt > 0, gmp_r[...], 0.0)
    pooled = pin_r[...] + jnp.concatenate([gmpv, gapv], axis=1)
    o = _lrelu(_dot(pooled, w1_r[...]) + b1_r[...])
    o = _lrelu(_dot(o, w2_r[...]) + b2_r[...])
    out_r[...] = _dot(o, w3_r[...]) + b3_r[...]


def _tc_fin(gap, gmp, cnt, pooled, w1, b1, w2, b2, w3, b3):
    return pl.pallas_call(
        _tc_fin_body,
        out_shape=jax.ShapeDtypeStruct((G, 1), jnp.float32),
    )(gap, gmp, cnt, pooled, w1, b1, w2, b2, w3, b3)


@jax.jit
def kernel(x, edge_index, edge_attr, batch,
           W0, b0, Wc0, bc0, Wc1, bc1, Wc2, bc2,
           g0, bt0, g1, bt1, g2, bt2, W1, b1, W2, b2, W3, b3):
    del edge_attr
    src = edge_index[0]
    dst = edge_index[1]
    pad = jnp.full((EPAD - E,), N, jnp.int32)
    srcp = jnp.concatenate([src, pad])
    dstp = jnp.concatenate([dst, pad])

    bcol = batch.reshape(N, 1)

    degp = _sc_deg(dstp)
    degp_t = degp.reshape(NW, NP).T

    h, dis = _tc_preA(x, degp_t, W0, b0.reshape(1, H))
    gs, cnt = _tc_preB(bcol, h)
    y = _tc_preC(h, bcol, dis, gs, cnt, Wc0)

    pooled = jnp.zeros((G, 2 * H), jnp.float32)
    per_layer = [
        (bc0, g0, bt0, Wc1),
        (bc1, g1, bt1, Wc2),
    ]
    for bc, g, bt, wnext in per_layer:
        zf = _sc_scat(y, srcp, dstp).reshape(2, NP, H)
        gcn, sg, s2 = _tc_m1(zf[0], zf[1], y, dis, bcol, bc.reshape(1, H))
        h2, gap, gmp = _tc_m2(gcn, bcol, sg, s2, cnt,
                              g.reshape(1, H), bt.reshape(1, H))
        y, pooled = _tc_m3(h2, bcol, dis, gap, gmp, cnt, pooled, wnext)

    zf = _sc_scat(y, srcp, dstp).reshape(2, NP, H)
    gcn, sg, s2 = _tc_m1(zf[0], zf[1], y, dis, bcol, bc2.reshape(1, H))
    h2, gap, gmp = _tc_m2(gcn, bcol, sg, s2, cnt,
                          g2.reshape(1, H), bt2.reshape(1, H))
    out = _tc_fin(gap, gmp, cnt, pooled,
                  W1, b1.reshape(1, 4 * H), W2, b2.reshape(1, 4 * H),
                  W3, b3.reshape(1, 1))
    return out
